# baseline (speedup 1.0000x reference)
import functools

import jax
import jax.numpy as jnp
from jax import lax
from jax.experimental import pallas as pl
from jax.experimental.pallas import tpu as pltpu
from jax.experimental.pallas import tpu_sc as plsc

B = 16384
D = 64
H = D // 2
C = 100000
NS = 16
CW = 16
MOM = 0.9

_f32 = jnp.float32
_i32 = jnp.int32


SPT_A = B // NS
RCH_A = SPT_A // 64


def _sc_a_body(l_hbm, samp_hbm, zc_hbm, ones_hbm,
               reptab_out, rep_out, cnt_out,
               cnt_sh,
               l_v, samp_v, rep_v, n_v, ones_v):
    t = lax.axis_index("s")
    base = t * SPT_A

    pltpu.sync_copy(l_hbm.at[pl.ds(t * RCH_A, RCH_A)], l_v)
    pltpu.sync_copy(samp_hbm.at[pl.ds(t * RCH_A, RCH_A)], samp_v)
    pltpu.sync_copy(ones_hbm, ones_v)
    pltpu.sync_copy(zc_hbm, n_v)
    for k in range(RCH_A):
        pltpu.sync_copy(n_v, cnt_sh.at[pl.ds(base + k * 64, 64)])

    for j in range(RCH_A):
        pltpu.sync_copy(samp_v.at[j], reptab_out.at[l_v.at[j]])

    plsc.subcore_barrier()

    for j in range(RCH_A):
        pltpu.sync_copy(reptab_out.at[l_v.at[j]], rep_v.at[j])
        pltpu.sync_copy(ones_v, cnt_sh.at[rep_v.at[j]], add=True)
    pltpu.sync_copy(rep_v, rep_out.at[pl.ds(t * RCH_A, RCH_A)])

    plsc.subcore_barrier()

    for k in range(RCH_A):
        sl = pl.ds(base + k * 64, 64)
        pltpu.sync_copy(cnt_sh.at[sl], n_v)
        pltpu.sync_copy(n_v, cnt_out.at[sl])


_mesh_a = plsc.VectorSubcoreMesh(
    core_axis_name="c", subcore_axis_name="s", num_cores=1)

_sc_a = functools.partial(
    pl.kernel,
    out_type=[
        jax.ShapeDtypeStruct((C,), _i32),
        jax.ShapeDtypeStruct((B // 64, 64), _i32),
        jax.ShapeDtypeStruct((B, CW), _f32),
    ],
    mesh=_mesh_a,
    scratch_types=[
        pltpu.VMEM_SHARED((B, CW), _f32),
        pltpu.VMEM((RCH_A, 64), _i32),
        pltpu.VMEM((RCH_A, 64), _i32),
        pltpu.VMEM((RCH_A, 64), _i32),
        pltpu.VMEM((64, CW), _f32),
        pltpu.VMEM((64, CW), _f32),
    ],
)(_sc_a_body)


SPT_B = B // (2 * NS)
RCH_B = SPT_B // 64


def _sc_b_body(xh0_hbm, xh1_hbm, l_hbm, rep_hbm, cc_hbm, zrow_hbm,
               acc0_out, acc1_out, ccg_out,
               acc_sh,
               l_v, rep_v, xa_v, cc_v):
    cid = lax.axis_index("c")
    sid = lax.axis_index("s")
    w = cid * NS + sid
    base = w * SPT_B
    sbase = sid * SPT_A

    pltpu.sync_copy(l_hbm.at[pl.ds(w * RCH_B, RCH_B)], l_v)
    pltpu.sync_copy(rep_hbm.at[pl.ds(w * RCH_B, RCH_B)], rep_v)

    pltpu.sync_copy(zrow_hbm, xa_v)
    for k in range(2 * RCH_B):
        pltpu.sync_copy(xa_v, acc_sh.at[pl.ds(sbase + k * 64, 64)])

    plsc.subcore_barrier()

    @pl.when(cid == 0)
    def _():
        for j in range(RCH_B):
            pltpu.sync_copy(xh0_hbm.at[pl.ds(base + j * 64, 64)], xa_v)
            pltpu.sync_copy(xa_v, acc_sh.at[rep_v.at[j]], add=True)

    @pl.when(cid == 1)
    def _():
        for j in range(RCH_B):
            pltpu.sync_copy(xh1_hbm.at[pl.ds(base + j * 64, 64)], xa_v)
            pltpu.sync_copy(xa_v, acc_sh.at[rep_v.at[j]], add=True)

    for k in range(RCH_B):
        for h in range(2):
            pltpu.sync_copy(cc_hbm.at[l_v.at[k, pl.ds(32 * h, 32)]], cc_v)
            pltpu.sync_copy(cc_v, ccg_out.at[pl.ds(base + k * 64 + 32 * h, 32)])

    plsc.subcore_barrier()

    @pl.when(cid == 0)
    def _():
        for k in range(2 * RCH_B):
            sl = pl.ds(sbase + k * 64, 64)
            pltpu.sync_copy(acc_sh.at[sl], xa_v)
            pltpu.sync_copy(xa_v, acc0_out.at[sl])

    @pl.when(cid == 1)
    def _():
        for k in range(2 * RCH_B):
            sl = pl.ds(sbase + k * 64, 64)
            pltpu.sync_copy(acc_sh.at[sl], xa_v)
            pltpu.sync_copy(xa_v, acc1_out.at[sl])


_mesh_b = plsc.VectorSubcoreMesh(
    core_axis_name="c", subcore_axis_name="s", num_cores=2)

_sc_b = functools.partial(
    pl.kernel,
    out_type=[
        jax.ShapeDtypeStruct((B, H), _f32),
        jax.ShapeDtypeStruct((B, H), _f32),
        jax.ShapeDtypeStruct((B, 2 * D), _f32),
    ],
    mesh=_mesh_b,
    scratch_types=[
        pltpu.VMEM_SHARED((B, H), _f32),
        pltpu.VMEM((RCH_B, 64), _i32),
        pltpu.VMEM((RCH_B, 64), _i32),
        pltpu.VMEM((64, H), _f32),
        pltpu.VMEM((32, 2 * D), _f32),
    ],
)(_sc_b_body)


_TC_STEPS = 16
_TC_ROWS = B // _TC_STEPS


def _tc_body(acc0_ref, acc1_ref, cnt_ref, cc_ref, out_ref):
    i = pl.program_id(0)
    n = cnt_ref[:, :1]
    m = n > 0.0
    safe_n = jnp.where(m, n, 1.0)
    q = (1.0 - MOM) / safe_n
    v0 = MOM * cc_ref[:, :H] + q * acc0_ref[...]
    v1 = MOM * cc_ref[:, H:D] + q * acc1_ref[...]
    cs0 = cc_ref[:, D:D + H]
    cs1 = cc_ref[:, D + H:]
    vv = jnp.sum(v0 * v0 + v1 * v1, axis=1, keepdims=True)
    vs = jnp.sum(v0 * cs0 + v1 * cs1, axis=1, keepdims=True)
    ss = jnp.sum(cs0 * cs0 + cs1 * cs1, axis=1, keepdims=True)
    d = 1.0 - 2.0 * vs * lax.rsqrt(vv) + ss
    sd = jnp.sum(jnp.where(m, d, 0.0))
    nr = jnp.sum(m.astype(_f32))

    row0 = lax.broadcasted_iota(_i32, (8, 128), 0) == 0
    col = lax.broadcasted_iota(_i32, (8, 128), 1)
    add = (jnp.where(row0 & (col == 0), sd, 0.0)
           + jnp.where(row0 & (col == 1), nr, 0.0))

    @pl.when(i == 0)
    def _():
        out_ref[...] = jnp.zeros((8, 128), _f32)

    out_ref[...] += add

    @pl.when(i == _TC_STEPS - 1)
    def _():
        tot = out_ref[...]
        sd_t = jnp.sum(jnp.where(row0 & (col == 0), tot, 0.0))
        nr_t = jnp.sum(jnp.where(row0 & (col == 1), tot, 0.0))
        out_ref[...] = jnp.broadcast_to(sd_t / nr_t, (8, 128))


_tc_call = pl.pallas_call(
    _tc_body,
    grid=(_TC_STEPS,),
    in_specs=[
        pl.BlockSpec((_TC_ROWS, H), lambda i: (i, 0)),
        pl.BlockSpec((_TC_ROWS, H), lambda i: (i, 0)),
        pl.BlockSpec((_TC_ROWS, CW), lambda i: (i, 0)),
        pl.BlockSpec((_TC_ROWS, 2 * D), lambda i: (i, 0)),
    ],
    out_specs=pl.BlockSpec((8, 128), lambda i: (0, 0)),
    out_shape=jax.ShapeDtypeStruct((8, 128), _f32),
)


def kernel(x, l, center_img, center_skt):
    l2 = l.reshape(B // 64, 64)
    samp2 = jnp.arange(B, dtype=_i32).reshape(B // 64, 64)
    zrow = jnp.zeros((64, H), _f32)
    zc = jnp.zeros((64, CW), _f32)
    ones = jnp.ones((64, CW), _f32)
    cc = jnp.concatenate([center_img, center_skt], axis=1)
    xh0 = x[:, :H]
    xh1 = x[:, H:]
    _, rep2, cnt = _sc_a(l2, samp2, zc, ones)
    acc0, acc1, ccg = _sc_b(xh0, xh1, l2, rep2, cc, zrow)
    out = _tc_call(acc0, acc1, cnt, ccg)
    return out[0, 0]


# speedup vs baseline: 1.0248x; 1.0248x over previous
import functools

import jax
import jax.numpy as jnp
from jax import lax
from jax.experimental import pallas as pl
from jax.experimental.pallas import tpu as pltpu
from jax.experimental.pallas import tpu_sc as plsc

B = 16384
D = 64
H = D // 2
C = 100000
NS = 16
CW = 16
MOM = 0.9

_f32 = jnp.float32
_i32 = jnp.int32


SPT_A = B // NS
RCH_A = SPT_A // 128


def _sc_a_body(l_hbm, samp_hbm, zc_hbm, ones_hbm,
               reptab_out, rep_out, cnt_out,
               cnt_sh,
               l_v, samp_v, rep_v, zc_v, ones_v, sem):
    t = lax.axis_index("s")
    base = t * SPT_A

    pltpu.sync_copy(l_hbm.at[pl.ds(t * RCH_A, RCH_A)], l_v)
    pltpu.sync_copy(samp_hbm.at[pl.ds(t * RCH_A, RCH_A)], samp_v)
    pltpu.sync_copy(ones_hbm, ones_v)
    pltpu.sync_copy(zc_hbm, zc_v)

    for h in range(2):
        pltpu.sync_copy(zc_v, cnt_sh.at[pl.ds(base + h * 512, 512)])
    for j in range(RCH_A):
        pltpu.sync_copy(samp_v.at[j], reptab_out.at[l_v.at[j]])

    plsc.subcore_barrier()

    for j in range(RCH_A):
        pltpu.sync_copy(reptab_out.at[l_v.at[j]], rep_v.at[j])
        pltpu.sync_copy(ones_v, cnt_sh.at[rep_v.at[j]], add=True)
    pltpu.sync_copy(rep_v, rep_out.at[pl.ds(t * RCH_A, RCH_A)])

    plsc.subcore_barrier()

    for h in range(2):
        sl = pl.ds(base + h * 512, 512)
        pltpu.sync_copy(cnt_sh.at[sl], zc_v)
        pltpu.sync_copy(zc_v, cnt_out.at[sl])


_mesh_a = plsc.VectorSubcoreMesh(
    core_axis_name="c", subcore_axis_name="s", num_cores=1)

_sc_a = functools.partial(
    pl.kernel,
    out_type=[
        jax.ShapeDtypeStruct((C,), _i32),
        jax.ShapeDtypeStruct((B // 128, 128), _i32),
        jax.ShapeDtypeStruct((B, CW), _f32),
    ],
    mesh=_mesh_a,
    scratch_types=[
        pltpu.VMEM_SHARED((B, CW), _f32),
        pltpu.VMEM((RCH_A, 128), _i32),
        pltpu.VMEM((RCH_A, 128), _i32),
        pltpu.VMEM((RCH_A, 128), _i32),
        pltpu.VMEM((512, CW), _f32),
        pltpu.VMEM((128, CW), _f32),
        pltpu.SemaphoreType.DMA,
    ],
)(_sc_a_body)


SPT_B = B // (2 * NS)
RCH_B = SPT_B // 128


def _sc_b_body(xh0_hbm, xh1_hbm, l_hbm, rep_hbm, cc_hbm, zrow_hbm,
               acc0_out, acc1_out, ccg_out,
               acc_sh,
               l_v, rep_v, x_v, cc_v, sem, sem2):
    cid = lax.axis_index("c")
    sid = lax.axis_index("s")
    w = cid * NS + sid
    base = w * SPT_B
    sbase = sid * SPT_A

    pltpu.sync_copy(l_hbm.at[pl.ds(w * RCH_B, RCH_B)], l_v)
    pltpu.sync_copy(rep_hbm.at[pl.ds(w * RCH_B, RCH_B)], rep_v)

    pltpu.sync_copy(zrow_hbm, x_v)
    for h in range(2):
        pltpu.sync_copy(x_v, acc_sh.at[pl.ds(sbase + h * 512, 512)])

    plsc.subcore_barrier()

    @pl.when(cid == 0)
    def _():
        pltpu.sync_copy(xh0_hbm.at[pl.ds(base, SPT_B)], x_v)

    @pl.when(cid == 1)
    def _():
        pltpu.sync_copy(xh1_hbm.at[pl.ds(base, SPT_B)], x_v)

    for j in range(RCH_B):
        pltpu.sync_copy(x_v.at[pl.ds(j * 128, 128)],
                        acc_sh.at[rep_v.at[j]], add=True)

    for k in range(RCH_B):
        pltpu.sync_copy(cc_hbm.at[l_v.at[k]], cc_v)
        pltpu.sync_copy(cc_v, ccg_out.at[pl.ds(base + k * 128, 128)])

    plsc.subcore_barrier()

    @pl.when(cid == 0)
    def _():
        for h in range(2):
            sl = pl.ds(sbase + h * 512, 512)
            pltpu.sync_copy(acc_sh.at[sl], x_v)
            pltpu.sync_copy(x_v, acc0_out.at[sl])

    @pl.when(cid == 1)
    def _():
        for h in range(2):
            sl = pl.ds(sbase + h * 512, 512)
            pltpu.sync_copy(acc_sh.at[sl], x_v)
            pltpu.sync_copy(x_v, acc1_out.at[sl])


_mesh_b = plsc.VectorSubcoreMesh(
    core_axis_name="c", subcore_axis_name="s", num_cores=2)

_sc_b = functools.partial(
    pl.kernel,
    out_type=[
        jax.ShapeDtypeStruct((B, H), _f32),
        jax.ShapeDtypeStruct((B, H), _f32),
        jax.ShapeDtypeStruct((B, 2 * D), _f32),
    ],
    mesh=_mesh_b,
    scratch_types=[
        pltpu.VMEM_SHARED((B, H), _f32),
        pltpu.VMEM((RCH_B, 128), _i32),
        pltpu.VMEM((RCH_B, 128), _i32),
        pltpu.VMEM((SPT_B, H), _f32),
        pltpu.VMEM((128, 2 * D), _f32),
        pltpu.SemaphoreType.DMA,
        pltpu.SemaphoreType.DMA,
    ],
)(_sc_b_body)


_TC_STEPS = 16
_TC_ROWS = B // _TC_STEPS


def _tc_body(acc0_ref, acc1_ref, cnt_ref, cc_ref, out_ref):
    i = pl.program_id(0)
    n = cnt_ref[:, :1]
    m = n > 0.0
    safe_n = jnp.where(m, n, 1.0)
    q = (1.0 - MOM) / safe_n
    v0 = MOM * cc_ref[:, :H] + q * acc0_ref[...]
    v1 = MOM * cc_ref[:, H:D] + q * acc1_ref[...]
    cs0 = cc_ref[:, D:D + H]
    cs1 = cc_ref[:, D + H:]
    vv = jnp.sum(v0 * v0 + v1 * v1, axis=1, keepdims=True)
    vs = jnp.sum(v0 * cs0 + v1 * cs1, axis=1, keepdims=True)
    ss = jnp.sum(cs0 * cs0 + cs1 * cs1, axis=1, keepdims=True)
    d = 1.0 - 2.0 * vs * lax.rsqrt(vv) + ss
    sd = jnp.sum(jnp.where(m, d, 0.0))
    nr = jnp.sum(m.astype(_f32))

    row0 = lax.broadcasted_iota(_i32, (8, 128), 0) == 0
    col = lax.broadcasted_iota(_i32, (8, 128), 1)
    add = (jnp.where(row0 & (col == 0), sd, 0.0)
           + jnp.where(row0 & (col == 1), nr, 0.0))

    @pl.when(i == 0)
    def _():
        out_ref[...] = jnp.zeros((8, 128), _f32)

    out_ref[...] += add

    @pl.when(i == _TC_STEPS - 1)
    def _():
        tot = out_ref[...]
        sd_t = jnp.sum(jnp.where(row0 & (col == 0), tot, 0.0))
        nr_t = jnp.sum(jnp.where(row0 & (col == 1), tot, 0.0))
        out_ref[...] = jnp.broadcast_to(sd_t / nr_t, (8, 128))


_tc_call = pl.pallas_call(
    _tc_body,
    grid=(_TC_STEPS,),
    in_specs=[
        pl.BlockSpec((_TC_ROWS, H), lambda i: (i, 0)),
        pl.BlockSpec((_TC_ROWS, H), lambda i: (i, 0)),
        pl.BlockSpec((_TC_ROWS, CW), lambda i: (i, 0)),
        pl.BlockSpec((_TC_ROWS, 2 * D), lambda i: (i, 0)),
    ],
    out_specs=pl.BlockSpec((8, 128), lambda i: (0, 0)),
    out_shape=jax.ShapeDtypeStruct((8, 128), _f32),
)


def kernel(x, l, center_img, center_skt):
    l2 = l.reshape(B // 128, 128)
    samp2 = jnp.arange(B, dtype=_i32).reshape(B // 128, 128)
    zrow = jnp.zeros((SPT_B, H), _f32)
    zc = jnp.zeros((512, CW), _f32)
    ones = jnp.ones((128, CW), _f32)
    cc = jnp.concatenate([center_img, center_skt], axis=1)
    xh0 = x[:, :H]
    xh1 = x[:, H:]
    _, rep2, cnt = _sc_a(l2, samp2, zc, ones)
    acc0, acc1, ccg = _sc_b(xh0, xh1, l2, rep2, cc, zrow)
    out = _tc_call(acc0, acc1, cnt, ccg)
    return out[0, 0]
